# baseline (speedup 1.0000x reference)
import jax
import jax.numpy as jnp
from jax.experimental import pallas as pl
from jax.experimental.pallas import tpu as pltpu

DIM = 4096
N_EXPERTS = 64
TOPK = 8
BT = 1024


def _gate_kernel(x_ref, w_ref, wout_ref, iout_ref):
    x = x_ref[...]
    w = w_ref[...]
    scores = jax.lax.dot_general(
        x, w, (((1,), (1,)), ((), ())), preferred_element_type=jnp.float32
    )
    m = jnp.max(scores, axis=-1, keepdims=True)
    e = jnp.exp(scores - m)
    probs = e / jnp.sum(e, axis=-1, keepdims=True)

    wout_ref[...] = probs[:, :TOPK]
    iout_ref[...] = jax.lax.broadcasted_iota(jnp.int32, (x.shape[0], TOPK), 1)
    return
    iota = jax.lax.broadcasted_iota(jnp.int32, probs.shape, 1)
    s = probs
    vals, idxs = [], []
    for _ in range(TOPK):
        mx = jnp.max(s, axis=-1, keepdims=True)
        idx = jnp.min(jnp.where(s == mx, iota, N_EXPERTS), axis=-1, keepdims=True)
        vals.append(mx)
        idxs.append(idx)
        s = jnp.where(iota == idx, -1.0, s)
    wout_ref[...] = jnp.concatenate(vals, axis=1)
    iout_ref[...] = jnp.concatenate(idxs, axis=1)


def kernel(x, weight):
    n_tokens = x.shape[0]
    grid = (n_tokens // BT,)
    wout, iout = pl.pallas_call(
        _gate_kernel,
        grid=grid,
        in_specs=[
            pl.BlockSpec((BT, DIM), lambda i: (i, 0)),
            pl.BlockSpec((N_EXPERTS, DIM), lambda i: (0, 0)),
        ],
        out_specs=[
            pl.BlockSpec((BT, TOPK), lambda i: (i, 0)),
            pl.BlockSpec((BT, TOPK), lambda i: (i, 0)),
        ],
        out_shape=[
            jax.ShapeDtypeStruct((n_tokens, TOPK), jnp.float32),
            jax.ShapeDtypeStruct((n_tokens, TOPK), jnp.int32),
        ],
    )(x, weight)
    return wout, iout


# speedup vs baseline: 1.1096x; 1.1096x over previous
import jax
import jax.numpy as jnp
from jax.experimental import pallas as pl
from jax.experimental.pallas import tpu as pltpu

DIM = 4096
N_EXPERTS = 64
TOPK = 8
BT = 512


def _gate_kernel(x_ref, w_ref, wout_ref, iout_ref):
    x = x_ref[...]
    w = w_ref[...]
    scores = jax.lax.dot_general(
        w, x, (((1,), (1,)), ((), ())), preferred_element_type=jnp.float32
    )
    m = jnp.max(scores, axis=0, keepdims=True)
    e = jnp.exp(scores - m)
    probs = e / jnp.sum(e, axis=0, keepdims=True)

    iota = jax.lax.broadcasted_iota(jnp.int32, probs.shape, 0)
    s = probs
    vals, idxs = [], []
    for k in range(TOPK):
        mx = jnp.max(s, axis=0, keepdims=True)
        idx = jnp.min(jnp.where(s == mx, iota, N_EXPERTS), axis=0, keepdims=True)
        vals.append(mx)
        idxs.append(idx)
        if k + 1 < TOPK:
            s = jnp.where(iota == idx, -1.0, s)
    wout_ref[...] = jnp.concatenate(vals, axis=0)
    iout_ref[...] = jnp.concatenate(idxs, axis=0)


def kernel(x, weight):
    n_tokens = x.shape[0]
    grid = (n_tokens // BT,)
    wout_t, iout_t = pl.pallas_call(
        _gate_kernel,
        grid=grid,
        in_specs=[
            pl.BlockSpec((BT, DIM), lambda i: (i, 0)),
            pl.BlockSpec((N_EXPERTS, DIM), lambda i: (0, 0)),
        ],
        out_specs=[
            pl.BlockSpec((TOPK, BT), lambda i: (0, i)),
            pl.BlockSpec((TOPK, BT), lambda i: (0, i)),
        ],
        out_shape=[
            jax.ShapeDtypeStruct((TOPK, n_tokens), jnp.float32),
            jax.ShapeDtypeStruct((TOPK, n_tokens), jnp.int32),
        ],
    )(x, weight)
    return wout_t.T, iout_t.T


# speedup vs baseline: 1.1957x; 1.0776x over previous
import jax
import jax.numpy as jnp
from jax.experimental import pallas as pl
from jax.experimental.pallas import tpu as pltpu

DIM = 4096
N_EXPERTS = 64
TOPK = 8
BT = 1024


def _gate_kernel(x_ref, w_ref, wout_ref, iout_ref):
    x = x_ref[...]
    w = w_ref[...]
    scores = jax.lax.dot_general(
        w, x, (((1,), (1,)), ((), ())), preferred_element_type=jnp.float32
    )
    m = jnp.max(scores, axis=0, keepdims=True)
    e = jnp.exp(scores - m)
    probs = e / jnp.sum(e, axis=0, keepdims=True)

    iota = jax.lax.broadcasted_iota(jnp.int32, probs.shape, 0)
    s = probs
    vals, idxs = [], []
    for k in range(TOPK):
        mx = jnp.max(s, axis=0, keepdims=True)
        idx = jnp.min(jnp.where(s == mx, iota, N_EXPERTS), axis=0, keepdims=True)
        vals.append(mx)
        idxs.append(idx)
        if k + 1 < TOPK:
            s = jnp.where(iota == idx, -1.0, s)
    wout_ref[...] = jnp.concatenate(vals, axis=0)
    iout_ref[...] = jnp.concatenate(idxs, axis=0)


def kernel(x, weight):
    n_tokens = x.shape[0]
    grid = (n_tokens // BT,)
    wout_t, iout_t = pl.pallas_call(
        _gate_kernel,
        grid=grid,
        in_specs=[
            pl.BlockSpec((BT, DIM), lambda i: (i, 0)),
            pl.BlockSpec((N_EXPERTS, DIM), lambda i: (0, 0)),
        ],
        out_specs=[
            pl.BlockSpec((TOPK, BT), lambda i: (0, i)),
            pl.BlockSpec((TOPK, BT), lambda i: (0, i)),
        ],
        out_shape=[
            jax.ShapeDtypeStruct((TOPK, n_tokens), jnp.float32),
            jax.ShapeDtypeStruct((TOPK, n_tokens), jnp.int32),
        ],
    )(x, weight)
    return wout_t.T, iout_t.T
